# Initial kernel scaffold; baseline (speedup 1.0000x reference)
import jax
import jax.numpy as jnp
from jax.experimental import pallas as pl


def kernel(barycenter_weights, codebook, node_distributions, batch_idx):
    raise NotImplementedError("write your pallas kernel here")


# speedup vs baseline: 2.4205x; 2.4205x over previous
import functools

import jax
import jax.numpy as jnp
from jax import lax
from jax.experimental import pallas as pl
from jax.experimental.pallas import tpu as pltpu
from jax.experimental.pallas import tpu_sc as plsc

B = 1024
K = 512
D = 256
N = 50000
S = 4

L = 16
NC = 2
NS = 16
NW = NC * NS

CH = 16
SEGW = B // NW

_mesh = plsc.VectorSubcoreMesh(core_axis_name="c", subcore_axis_name="s")


@functools.partial(
    pl.kernel,
    mesh=_mesh,
    out_type=jax.ShapeDtypeStruct((B, D), jnp.float32),
    scratch_types=[
        pltpu.VMEM((N + L,), jnp.int32),
        pltpu.VMEM((CH, S, D), jnp.float32),
        pltpu.VMEM((SEGW, D), jnp.float32),
        pltpu.VMEM((SEGW, L), jnp.float32),
        pltpu.VMEM((SEGW, D), jnp.float32),
    ],
)
def _sc_segment_mean(nd_hbm, bi_hbm, out_hbm, bi_v, buf, acc, cnt, trad):
    cid = lax.axis_index("c")
    sid = lax.axis_index("s")
    wid = sid * NC + cid
    base = wid * SEGW

    pltpu.sync_copy(bi_hbm, bi_v.at[pl.ds(0, N)])

    for r in range(SEGW):
        for k in range(D // L):
            acc[r, pl.ds(k * L, L)] = jnp.zeros((L,), jnp.float32)
        cnt[r, :] = jnp.zeros((L,), jnp.float32)

    def lower_bound(t):
        pos = jnp.int32(0)
        for sh in range(15, -1, -1):
            nxt = pos + (1 << sh)
            probe = bi_v[pl.ds(jnp.minimum(nxt - 1, N - 1), L)][0]
            ok = (nxt <= N) & (probe < t)
            pos = jnp.where(ok, nxt, pos)
        return pos

    lo = lower_bound(base)
    hi = lower_bound(base + SEGW)
    m0 = lax.div(lo, CH)
    m1 = lax.div(hi + (CH - 1), CH)

    one_vec = jnp.ones((L,), jnp.float32)

    def chunk_body(m, carry):
        node0 = pl.multiple_of(m * CH, CH)
        pltpu.sync_copy(nd_hbm.at[pl.ds(node0, CH)], buf)
        ids = bi_v[pl.ds(node0, CH)]
        for i in range(CH):
            gi = node0 + i

            @pl.when((gi >= lo) & (gi < hi))
            def _():
                r = ids[i] - base
                for k in range(D // L):
                    sl = pl.ds(k * L, L)
                    acc[r, sl] += ((buf[i, 0, sl] + buf[i, 1, sl])
                                   + (buf[i, 2, sl] + buf[i, 3, sl]))
                cnt[r, :] += one_vec

        return carry

    lax.fori_loop(m0, m1, chunk_body, 0)

    for r in range(SEGW):
        dv = jnp.maximum(cnt[r, :] * float(S), 1.0)
        for k in range(D // L):
            sl = pl.ds(k * L, L)
            trad[r, sl] = acc[r, sl] / dv
    pltpu.sync_copy(trad, out_hbm.at[pl.ds(base, SEGW)])


def _tc_matmul_body(bw_ref, cb_ref, o_ref):
    o_ref[...] = jnp.dot(bw_ref[...], cb_ref[...],
                         preferred_element_type=jnp.float32)


_tc_matmul = pl.pallas_call(
    _tc_matmul_body,
    out_shape=jax.ShapeDtypeStruct((B, D), jnp.float32),
)


def kernel(barycenter_weights, codebook, node_distributions, batch_idx):
    bi = batch_idx.astype(jnp.int32)
    trad = _sc_segment_mean(node_distributions, bi)
    mm = _tc_matmul(barycenter_weights, codebook)
    return jnp.concatenate([mm, trad], axis=1)


# speedup vs baseline: 2.8545x; 1.1793x over previous
import functools

import jax
import jax.numpy as jnp
from jax import lax
from jax.experimental import pallas as pl
from jax.experimental.pallas import tpu as pltpu
from jax.experimental.pallas import tpu_sc as plsc

B = 1024
K = 512
D = 256
N = 50000
S = 4

L = 16
NC = 2
NS = 16
NW = NC * NS

CH = 16
SEGW = B // NW

_mesh = plsc.VectorSubcoreMesh(core_axis_name="c", subcore_axis_name="s")


@functools.partial(
    pl.kernel,
    mesh=_mesh,
    out_type=jax.ShapeDtypeStruct((B, D), jnp.float32),
    scratch_types=[
        pltpu.VMEM((N + L,), jnp.int32),
        pltpu.VMEM((CH, S, D), jnp.float32),
        pltpu.VMEM((CH, S, D), jnp.float32),
        pltpu.VMEM((SEGW, D), jnp.float32),
        pltpu.VMEM((SEGW, L), jnp.float32),
        pltpu.VMEM((SEGW, D), jnp.float32),
        pltpu.SemaphoreType.DMA,
        pltpu.SemaphoreType.DMA,
    ],
)
def _sc_segment_mean(nd_hbm, bi_hbm, out_hbm, bi_v, buf_a, buf_b, acc, cnt,
                     trad, sem_a, sem_b):
    cid = lax.axis_index("c")
    sid = lax.axis_index("s")
    wid = sid * NC + cid
    base = wid * SEGW

    pltpu.sync_copy(bi_hbm, bi_v.at[pl.ds(0, N)])

    for r in range(SEGW):
        for k in range(D // L):
            acc[r, pl.ds(k * L, L)] = jnp.zeros((L,), jnp.float32)
        cnt[r, :] = jnp.zeros((L,), jnp.float32)

    def lower_bound(t):
        pos = jnp.int32(0)
        for sh in range(15, -1, -1):
            nxt = pos + (1 << sh)
            probe = bi_v[pl.ds(jnp.minimum(nxt - 1, N - 1), L)][0]
            ok = (nxt <= N) & (probe < t)
            pos = jnp.where(ok, nxt, pos)
        return pos

    lo = lower_bound(base)
    hi = lower_bound(base + SEGW)
    m0 = lax.div(lo, CH)
    m1 = lax.div(hi + (CH - 1), CH)

    one_vec = jnp.ones((L,), jnp.float32)

    def start(m, buf, sem):
        pltpu.async_copy(nd_hbm.at[pl.ds(pl.multiple_of(m * CH, CH), CH)],
                         buf, sem)

    def wait(buf, sem):
        pltpu.make_async_copy(nd_hbm.at[pl.ds(0, CH)], buf, sem).wait()

    def process(m, buf):
        node0 = pl.multiple_of(m * CH, CH)
        ids = bi_v[pl.ds(node0, CH)]
        for i in range(CH):
            gi = node0 + i

            @pl.when((gi >= lo) & (gi < hi))
            def _():
                r = ids[i] - base
                for k in range(D // L):
                    sl = pl.ds(k * L, L)
                    acc[r, sl] += ((buf[i, 0, sl] + buf[i, 1, sl])
                                   + (buf[i, 2, sl] + buf[i, 3, sl]))
                cnt[r, :] += one_vec

    @pl.when(m0 < m1)
    def _():
        start(m0, buf_a, sem_a)

    def pair_body(g, carry):
        m_a = m0 + 2 * g
        m_b = m_a + 1

        @pl.when(m_b < m1)
        def _():
            start(m_b, buf_b, sem_b)

        wait(buf_a, sem_a)
        process(m_a, buf_a)

        @pl.when(m_a + 2 < m1)
        def _():
            start(m_a + 2, buf_a, sem_a)

        @pl.when(m_b < m1)
        def _():
            wait(buf_b, sem_b)
            process(m_b, buf_b)

        return carry

    lax.fori_loop(0, lax.div(m1 - m0 + 1, 2), pair_body, 0)

    for r in range(SEGW):
        dv = jnp.maximum(cnt[r, :] * float(S), 1.0)
        for k in range(D // L):
            sl = pl.ds(k * L, L)
            trad[r, sl] = acc[r, sl] / dv
    pltpu.sync_copy(trad, out_hbm.at[pl.ds(base, SEGW)])


def _tc_matmul_body(bw_ref, cb_ref, o_ref):
    o_ref[...] = jnp.dot(bw_ref[...], cb_ref[...],
                         preferred_element_type=jnp.float32)


_tc_matmul = pl.pallas_call(
    _tc_matmul_body,
    out_shape=jax.ShapeDtypeStruct((B, D), jnp.float32),
)


def kernel(barycenter_weights, codebook, node_distributions, batch_idx):
    bi = batch_idx.astype(jnp.int32)
    trad = _sc_segment_mean(node_distributions, bi)
    mm = _tc_matmul(barycenter_weights, codebook)
    return jnp.concatenate([mm, trad], axis=1)


# speedup vs baseline: 2.9685x; 1.0399x over previous
import functools

import jax
import jax.numpy as jnp
from jax import lax
from jax.experimental import pallas as pl
from jax.experimental.pallas import tpu as pltpu
from jax.experimental.pallas import tpu_sc as plsc

B = 1024
K = 512
D = 256
N = 50000
S = 4

L = 16
NC = 2
NS = 16
NW = NC * NS

CH = 16
SEGW = B // NW

_mesh = plsc.VectorSubcoreMesh(core_axis_name="c", subcore_axis_name="s")


@functools.partial(
    pl.kernel,
    mesh=_mesh,
    out_type=jax.ShapeDtypeStruct((B, D), jnp.float32),
    scratch_types=[
        pltpu.VMEM((N + L,), jnp.int32),
        pltpu.VMEM((CH, S, D), jnp.float32),
        pltpu.VMEM((CH, S, D), jnp.float32),
        pltpu.VMEM((SEGW + 1, D), jnp.float32),
        pltpu.VMEM((SEGW + 1, L), jnp.float32),
        pltpu.VMEM((SEGW, D), jnp.float32),
        pltpu.SemaphoreType.DMA,
        pltpu.SemaphoreType.DMA,
    ],
)
def _sc_segment_mean(nd_hbm, bi_hbm, out_hbm, bi_v, buf_a, buf_b, acc, cnt,
                     trad, sem_a, sem_b):
    cid = lax.axis_index("c")
    sid = lax.axis_index("s")
    wid = sid * NC + cid
    base = wid * SEGW

    pltpu.sync_copy(bi_hbm, bi_v.at[pl.ds(0, N)])

    for r in range(SEGW + 1):
        for k in range(D // L):
            acc[r, pl.ds(k * L, L)] = jnp.zeros((L,), jnp.float32)
        cnt[r, :] = jnp.zeros((L,), jnp.float32)

    def lower_bound(t):
        pos = jnp.int32(0)
        for sh in range(15, -1, -1):
            nxt = pos + (1 << sh)
            probe = bi_v[pl.ds(jnp.minimum(nxt - 1, N - 1), L)][0]
            ok = (nxt <= N) & (probe < t)
            pos = jnp.where(ok, nxt, pos)
        return pos

    lo = lower_bound(base)
    hi = lower_bound(base + SEGW)
    m0 = lax.div(lo, CH)
    m1 = lax.div(hi + (CH - 1), CH)

    one_vec = jnp.ones((L,), jnp.float32)

    def start(m, buf, sem):
        pltpu.async_copy(nd_hbm.at[pl.ds(pl.multiple_of(m * CH, CH), CH)],
                         buf, sem)

    def wait(buf, sem):
        pltpu.make_async_copy(nd_hbm.at[pl.ds(0, CH)], buf, sem).wait()

    def process(m, buf):
        node0 = pl.multiple_of(m * CH, CH)
        ids = bi_v[pl.ds(node0, CH)]
        for i in range(CH):
            gi = node0 + i
            in_range = (gi >= lo) & (gi < hi)
            r = jnp.where(in_range, ids[i] - base, SEGW)
            for k in range(D // L):
                sl = pl.ds(k * L, L)
                acc[r, sl] += ((buf[i, 0, sl] + buf[i, 1, sl])
                               + (buf[i, 2, sl] + buf[i, 3, sl]))
            cnt[r, :] += one_vec

    @pl.when(m0 < m1)
    def _():
        start(m0, buf_a, sem_a)

    def pair_body(g, carry):
        m_a = m0 + 2 * g
        m_b = m_a + 1

        @pl.when(m_b < m1)
        def _():
            start(m_b, buf_b, sem_b)

        wait(buf_a, sem_a)
        process(m_a, buf_a)

        @pl.when(m_a + 2 < m1)
        def _():
            start(m_a + 2, buf_a, sem_a)

        @pl.when(m_b < m1)
        def _():
            wait(buf_b, sem_b)
            process(m_b, buf_b)

        return carry

    lax.fori_loop(0, lax.div(m1 - m0 + 1, 2), pair_body, 0)

    for r in range(SEGW):
        dv = jnp.maximum(cnt[r, :] * float(S), 1.0)
        for k in range(D // L):
            sl = pl.ds(k * L, L)
            trad[r, sl] = acc[r, sl] / dv
    pltpu.sync_copy(trad, out_hbm.at[pl.ds(base, SEGW)])


def _tc_matmul_body(bw_ref, cb_ref, o_ref):
    o_ref[...] = jnp.dot(bw_ref[...], cb_ref[...],
                         preferred_element_type=jnp.float32)


_tc_matmul = pl.pallas_call(
    _tc_matmul_body,
    out_shape=jax.ShapeDtypeStruct((B, D), jnp.float32),
)


def kernel(barycenter_weights, codebook, node_distributions, batch_idx):
    bi = batch_idx.astype(jnp.int32)
    trad = _sc_segment_mean(node_distributions, bi)
    mm = _tc_matmul(barycenter_weights, codebook)
    return jnp.concatenate([mm, trad], axis=1)


# speedup vs baseline: 4.8961x; 1.6494x over previous
import functools

import jax
import jax.numpy as jnp
from jax import lax
from jax.experimental import pallas as pl
from jax.experimental.pallas import tpu as pltpu
from jax.experimental.pallas import tpu_sc as plsc

B = 1024
K = 512
D = 256
N = 50000
S = 4

L = 16
NC = 2
NS = 16
NW = NC * NS

CH = 16
NCHUNKS = N // CH
SEGW = B // NW

_mesh = plsc.VectorSubcoreMesh(core_axis_name="c", subcore_axis_name="s")


@functools.partial(
    pl.kernel,
    mesh=_mesh,
    out_type=jax.ShapeDtypeStruct((B, D), jnp.float32),
    scratch_types=[
        pltpu.VMEM((N + L,), jnp.int32),
        pltpu.VMEM((CH, S, D), jnp.float32),
        pltpu.VMEM((CH, S, D), jnp.float32),
        pltpu.VMEM((SEGW + 1, D), jnp.float32),
        pltpu.VMEM((SEGW + 1, L), jnp.float32),
        pltpu.VMEM((SEGW, D), jnp.float32),
        pltpu.SemaphoreType.DMA,
        pltpu.SemaphoreType.DMA,
    ],
)
def _sc_segment_mean(nd_hbm, bi_hbm, out_hbm, bi_v, buf_a, buf_b, acc, cnt,
                     trad, sem_a, sem_b):
    cid = lax.axis_index("c")
    sid = lax.axis_index("s")
    wid = sid * NC + cid
    base = wid * SEGW

    pltpu.sync_copy(bi_hbm, bi_v.at[pl.ds(0, N)])

    for r in range(SEGW + 1):
        for k in range(D // L):
            acc[r, pl.ds(k * L, L)] = jnp.zeros((L,), jnp.float32)
        cnt[r, :] = jnp.zeros((L,), jnp.float32)

    def lower_bound(t):
        pos = jnp.int32(0)
        for sh in range(15, -1, -1):
            nxt = pos + (1 << sh)
            probe = bi_v[pl.ds(jnp.minimum(nxt - 1, N - 1), L)][0]
            ok = (nxt <= N) & (probe < t)
            pos = jnp.where(ok, nxt, pos)
        return pos

    lo = lower_bound(base)
    hi = lower_bound(base + SEGW)
    m0 = lax.div(lo, CH)
    m1 = lax.div(hi + (CH - 1), CH)

    npairs = lax.div(m1 - m0 + 1, 2)

    def clampm(m):
        return jnp.minimum(m, NCHUNKS - 1)

    def start(m, buf, sem):
        pltpu.async_copy(nd_hbm.at[pl.ds(pl.multiple_of(m * CH, CH), CH)],
                         buf, sem)

    def wait(buf, sem):
        pltpu.make_async_copy(nd_hbm.at[pl.ds(0, CH)], buf, sem).wait()

    def flush(pred, run_id, run_cnt, regs):
        @pl.when(pred)
        def _():
            ok = (run_id >= base) & (run_id < base + SEGW)
            r = jnp.where(ok, run_id - base, SEGW)
            for k in range(D // L):
                acc[r, pl.ds(k * L, L)] += regs[k]
            cnt[r, :] += jnp.broadcast_to(run_cnt, (L,))

    def process(m, buf, st):
        run_id, run_cnt, regs = st
        node0 = pl.multiple_of(m * CH, CH)
        ids = bi_v[pl.ds(clampm(m) * CH, CH)]
        for i in range(CH):
            gi = node0 + i
            in_range = (gi >= lo) & (gi < hi)
            nid = jnp.where(in_range, ids[i], jnp.int32(-1))
            change = nid != run_id
            flush(change, run_id, run_cnt, regs)
            new_regs = []
            for k in range(D // L):
                sl = pl.ds(k * L, L)
                s = ((buf[i, 0, sl] + buf[i, 1, sl])
                     + (buf[i, 2, sl] + buf[i, 3, sl]))
                new_regs.append(jnp.where(change, s, regs[k] + s))
            regs = new_regs
            run_cnt = jnp.where(change, jnp.float32(1.0), run_cnt + 1.0)
            run_id = nid
        return run_id, run_cnt, regs

    @pl.when(npairs > 0)
    def _():
        start(m0, buf_a, sem_a)

    def pair_body(g, st):
        m_a = m0 + 2 * g
        m_b = m_a + 1
        start(clampm(m_b), buf_b, sem_b)
        wait(buf_a, sem_a)
        st = process(m_a, buf_a, st)

        @pl.when(g + 1 < npairs)
        def _():
            start(clampm(m_a + 2), buf_a, sem_a)

        wait(buf_b, sem_b)
        st = process(m_b, buf_b, st)
        return st

    st0 = (jnp.int32(-1), jnp.float32(0.0),
           [jnp.zeros((L,), jnp.float32) for _ in range(D // L)])
    run_id, run_cnt, regs = lax.fori_loop(0, npairs, pair_body, st0)
    flush(run_id >= 0, run_id, run_cnt, regs)

    for r in range(SEGW):
        dv = jnp.maximum(cnt[r, :] * float(S), 1.0)
        for k in range(D // L):
            sl = pl.ds(k * L, L)
            trad[r, sl] = acc[r, sl] / dv
    pltpu.sync_copy(trad, out_hbm.at[pl.ds(base, SEGW)])


def _tc_matmul_body(bw_ref, cb_ref, o_ref):
    o_ref[...] = jnp.dot(bw_ref[...], cb_ref[...],
                         preferred_element_type=jnp.float32)


_tc_matmul = pl.pallas_call(
    _tc_matmul_body,
    out_shape=jax.ShapeDtypeStruct((B, D), jnp.float32),
)


def kernel(barycenter_weights, codebook, node_distributions, batch_idx):
    bi = batch_idx.astype(jnp.int32)
    trad = _sc_segment_mean(node_distributions, bi)
    mm = _tc_matmul(barycenter_weights, codebook)
    return jnp.concatenate([mm, trad], axis=1)
